# baseline (speedup 1.0000x reference)
import jax
import jax.numpy as jnp
from jax import lax
from jax.experimental import pallas as pl
from jax.experimental.pallas import tpu as pltpu
from jax.experimental.pallas import tpu_sc as plsc

_STEP = 1.0
_N = 4096
_L = 16
_NC = 2
_NS = 16
_NW = _NC * _NS
_PAD = _N + _L


def _f16_rne(x_f32):
    bits = lax.bitcast_convert_type(x_f32, jnp.int32)
    r = (bits + ((bits >> 13) & 1) + 0xFFF) & ~0x1FFF
    return lax.bitcast_convert_type(r, jnp.float32)


def _body(p_hbm, a_hbm, out_hbm, p_v, a_v, acc_v):
    cid = lax.axis_index("c")
    sid = lax.axis_index("s")
    wid = sid * _NC + cid

    pltpu.sync_copy(p_hbm, p_v.at[pl.ds(0, _N)])
    p_v[pl.ds(_N, _L)] = jnp.zeros((_L,), jnp.float32)
    pltpu.sync_copy(a_hbm, a_v)
    av = a_v[...]

    c02 = jnp.float32(0.2)
    c08 = jnp.float32(0.8)
    zero = jnp.zeros((_L,), jnp.float32)
    lanes = lax.iota(jnp.int32, _L)

    def diag_body(m, acc):
        k = wid + _NW * m
        lenk = _N - k
        t_full = lenk >> 4
        rem = lenk & 15
        kf = jnp.full((_L,), k, dtype=jnp.int32).astype(jnp.float32)
        s = _f16_rne(kf * av)
        t02 = s * c02
        t08 = s * c08

        def contrib(a, b):
            raw = a - b
            x = jnp.where(raw >= zero, raw - t02, raw)
            y = jnp.where(x >= zero, jnp.maximum(x - t08, zero), x)
            return jnp.abs(y)

        def chunk(t, a2):
            off = t * _L
            return a2 + contrib(p_v[pl.ds(k + off, _L)], p_v[pl.ds(off, _L)])

        acc = lax.fori_loop(0, t_full, chunk, acc)

        off = t_full * _L
        y = contrib(p_v[pl.ds(k + off, _L)], p_v[pl.ds(off, _L)])
        y = jnp.where(lanes < jnp.full((_L,), rem, jnp.int32), y, zero)
        return acc + y

    acc = lax.fori_loop(0, _N // _NW, diag_body, jnp.zeros((_L,), jnp.float32))
    acc_v[...] = acc
    pltpu.sync_copy(acc_v, out_hbm.at[wid])


def kernel(predictions, z_spacing, nth_slice):
    p = predictions[:, 0]
    a_val = jnp.float32(_STEP) * z_spacing * nth_slice
    a_vec = jnp.full((_L,), a_val, dtype=jnp.float32)
    mesh = plsc.VectorSubcoreMesh(core_axis_name="c", subcore_axis_name="s")
    fn = pl.kernel(
        _body,
        out_type=jax.ShapeDtypeStruct((_NW, _L), jnp.float32),
        mesh=mesh,
        scratch_types=[
            pltpu.VMEM((_PAD,), jnp.float32),
            pltpu.VMEM((_L,), jnp.float32),
            pltpu.VMEM((_L,), jnp.float32),
        ],
    )
    partial = fn(p.astype(jnp.float32), a_vec)
    return jnp.sum(partial) / jnp.float32(_N * _N)


# speedup vs baseline: 1.7312x; 1.7312x over previous
import jax
import jax.numpy as jnp
from jax import lax
from jax.experimental import pallas as pl
from jax.experimental.pallas import tpu as pltpu
from jax.experimental.pallas import tpu_sc as plsc

_STEP = 1.0
_N = 4096
_L = 16
_NC = 2
_NS = 16
_NW = _NC * _NS
_G = 4
_NT = _N // (_G * _NW)
_PAD = _N + _L


def _f16_rne(x_f32):
    bits = lax.bitcast_convert_type(x_f32, jnp.int32)
    r = (bits + ((bits >> 13) & 1) + 0xFFF) & ~0x1FFF
    return lax.bitcast_convert_type(r, jnp.float32)


def _body(p_hbm, a_hbm, out_hbm, p_v, a_v, acc_v):
    cid = lax.axis_index("c")
    sid = lax.axis_index("s")
    wid = sid * _NC + cid

    pltpu.sync_copy(p_hbm, p_v.at[pl.ds(0, _N)])
    p_v[pl.ds(_N, _L)] = jnp.zeros((_L,), jnp.float32)
    pltpu.sync_copy(a_hbm, a_v)
    av = a_v[...]

    c02 = jnp.float32(0.2)
    c08 = jnp.float32(0.8)
    zero = jnp.zeros((_L,), jnp.float32)
    lanes = lax.iota(jnp.int32, _L)

    def contrib(a, b, t02, t08):
        raw = a - b
        x = jnp.where(raw >= zero, raw - t02, raw)
        return jnp.maximum(jnp.maximum(x - t08, -x), zero)

    def group_body(t, accs):
        gw = wid ^ ((t & 1) * (_NW - 1))
        k0 = (t * _NW + gw) * _G
        steps = []
        for u in range(_G):
            kf = jnp.full((_L,), k0 + u, dtype=jnp.int32).astype(jnp.float32)
            s = _f16_rne(kf * av)
            steps.append((s * c02, s * c08))
        t_min = (_N - (k0 + _G - 1)) >> 4

        def chunk(tc, accs4):
            off = tc * _L
            b = p_v[pl.ds(off, _L)]
            return tuple(
                accs4[u]
                + contrib(p_v[pl.ds(k0 + u + off, _L)], b, *steps[u])
                for u in range(_G)
            )

        accs = plsc.parallel_loop(0, t_min, carry=accs)(chunk)

        out = []
        for u in range(_G):
            lenk = _N - (k0 + u)
            t_u = lenk >> 4
            rem = lenk & 15

            def chunk_u(tc, a1, _u=u):
                off = tc * _L
                return a1 + contrib(
                    p_v[pl.ds(k0 + _u + off, _L)], p_v[pl.ds(off, _L)],
                    *steps[_u])

            a1 = lax.fori_loop(t_min, t_u, chunk_u, accs[u])
            off = t_u * _L
            y = contrib(p_v[pl.ds(k0 + u + off, _L)], p_v[pl.ds(off, _L)],
                        *steps[u])
            y = jnp.where(lanes < jnp.full((_L,), rem, jnp.int32), y, zero)
            out.append(a1 + y)
        return tuple(out)

    accs = lax.fori_loop(0, _NT, group_body, (zero,) * _G)
    acc_v[...] = accs[0] + accs[1] + accs[2] + accs[3]
    pltpu.sync_copy(acc_v, out_hbm.at[wid])


def kernel(predictions, z_spacing, nth_slice):
    p = predictions[:, 0]
    a_val = jnp.float32(_STEP) * z_spacing * nth_slice
    a_vec = jnp.full((_L,), a_val, dtype=jnp.float32)
    mesh = plsc.VectorSubcoreMesh(core_axis_name="c", subcore_axis_name="s")
    fn = pl.kernel(
        _body,
        out_type=jax.ShapeDtypeStruct((_NW, _L), jnp.float32),
        mesh=mesh,
        scratch_types=[
            pltpu.VMEM((_PAD,), jnp.float32),
            pltpu.VMEM((_L,), jnp.float32),
            pltpu.VMEM((_L,), jnp.float32),
        ],
    )
    partial = fn(p.astype(jnp.float32), a_vec)
    return jnp.sum(partial) / jnp.float32(_N * _N)
